# baseline (speedup 1.0000x reference)
import functools

import jax
import jax.numpy as jnp
from jax.experimental import pallas as pl
from jax.experimental.pallas import tpu as pltpu

B, N = 32, 512
NT = N // 128


def _fused_kernel(cf_ref, tour_ref, tnext_ref, q_ref, p_ref,
                  yp_ref, xev_ref, lsum_ref):
    b = pl.program_id(0)

    wc0 = p_ref[0]
    wc1 = p_ref[1]
    emb1 = p_ref[2]
    we0 = p_ref[3]
    we1 = p_ref[4]
    be0 = p_ref[5]
    be1 = p_ref[6]

    cf = cf_ref[0, :, :]
    qm = q_ref[0]

    xy2 = jnp.dot(cf, qm, preferred_element_type=jnp.float32)
    xy = xy2[0:1, :] + xy2[1:2, :]
    x_row = xy[:, :N]
    y_row = xy[:, N:]
    xcol = jnp.transpose(x_row)
    ycol = jnp.transpose(y_row)

    dx = xcol - x_row
    dy = ycol - y_row
    d = jnp.sqrt(dx * dx + dy * dy)
    xev_ref[0, :, :] = d

    s_row = x_row * wc0 + y_row * wc1 + emb1
    st_col = xcol * wc0 + ycol * wc1 + emb1
    base = st_col + s_row
    a0 = d * we0 + be0
    a1 = d * we1 + be1
    p0 = a0 + base
    p1 = a1 + base
    for t in range(NT):
        yp_ref[0, :, 2 * t, :] = p0[:, 128 * t:128 * (t + 1)]
        yp_ref[0, :, 2 * t + 1, :] = p1[:, 128 * t:128 * (t + 1)]

    tour = tour_ref[0, 0, :]
    tnext = tnext_ref[0, 0, :]
    col = jax.lax.broadcasted_iota(jnp.int32, (N, N), 1)
    a_oh = (tour[:, None] == col).astype(jnp.bfloat16)
    bn_oh = (tnext[:, None] == col).astype(jnp.bfloat16)
    dn = (((0,), (0,)), ((), ()))
    m_fwd = jax.lax.dot_general(a_oh, bn_oh, dn,
                                preferred_element_type=jnp.float32)
    m_bwd = jax.lax.dot_general(bn_oh, a_oh, dn,
                                preferred_element_type=jnp.float32)
    mask = (m_fwd + m_bwd) > 0.0

    mx = jnp.maximum(a0, a1)
    lse = mx + jnp.log1p(jnp.exp(jnp.minimum(a0, a1) - mx))
    sel = jnp.where(mask, a1, a0) - lse
    lsum_ref[0, 0, 0] = jnp.sum(sel)


@functools.partial(jax.jit, static_argnames=("interpret",))
def kernel(x_nodes_coord, y_tour, w_coord, emb, w_e, b_e, interpret=False):
    cf = x_nodes_coord.reshape(B, 1, 2 * N)
    cf_bits = jax.lax.bitcast_convert_type(cf, jnp.uint32)
    cf_hi = jax.lax.bitcast_convert_type(
        cf_bits & jnp.uint32(0xFFFF0000), jnp.float32)
    cf2 = jnp.concatenate([cf_hi, cf - cf_hi], axis=1)
    tour = y_tour.reshape(B, 1, N)
    tnext = jnp.roll(y_tour, -1, axis=-1).reshape(B, 1, N)
    c2 = jnp.arange(2 * N, dtype=jnp.int32)
    jn = jnp.arange(N, dtype=jnp.int32)
    q = jnp.concatenate(
        [(c2[:, None] == 2 * jn[None, :]),
         (c2[:, None] == 2 * jn[None, :] + 1)], axis=1)
    q = q.astype(jnp.float32).reshape(1, 2 * N, 2 * N)
    params = jnp.stack([w_coord[0], w_coord[1], emb[1],
                        w_e[0], w_e[1], b_e[0], b_e[1]])

    yp, xev, lsum = pl.pallas_call(
        _fused_kernel,
        grid=(B,),
        in_specs=[pl.BlockSpec((1, 2, 2 * N), lambda b: (b, 0, 0)),
                  pl.BlockSpec((1, 1, N), lambda b: (b, 0, 0)),
                  pl.BlockSpec((1, 1, N), lambda b: (b, 0, 0)),
                  pl.BlockSpec((1, 2 * N, 2 * N), lambda b: (0, 0, 0)),
                  pl.BlockSpec(memory_space=pltpu.SMEM)],
        out_specs=[
            pl.BlockSpec((1, N, 2 * NT, 128), lambda b: (b, 0, 0, 0)),
            pl.BlockSpec((1, N, N), lambda b: (b, 0, 0)),
            pl.BlockSpec((1, 1, 1), lambda b: (b, 0, 0),
                         memory_space=pltpu.SMEM),
        ],
        out_shape=[
            jax.ShapeDtypeStruct((B, N, 2 * NT, 128), jnp.float32),
            jax.ShapeDtypeStruct((B, N, N), jnp.float32),
            jax.ShapeDtypeStruct((B, 1, 1), jnp.float32),
        ],
        compiler_params=pltpu.CompilerParams(
            dimension_semantics=("parallel",)),
        interpret=interpret,
    )(cf2, tour, tnext, q, params)

    y_preds = (yp.reshape(B, N, NT, 2, 128)
               .transpose(0, 1, 2, 4, 3)
               .reshape(B, N, N, 2))
    loss = -jnp.sum(lsum) / jnp.float32(B * N * N)
    return (y_preds, loss, xev)


# speedup vs baseline: 1.0295x; 1.0295x over previous
import functools

import jax
import jax.numpy as jnp
from jax.experimental import pallas as pl
from jax.experimental.pallas import tpu as pltpu

B, N = 32, 512
NT = N // 128


def _fused_kernel(cf_ref, tour_ref, tnext_ref, q_ref, p_ref,
                  yp_ref, xev_ref, lsum_ref):
    b = pl.program_id(0)

    wc0 = p_ref[0]
    wc1 = p_ref[1]
    emb1 = p_ref[2]
    we0 = p_ref[3]
    we1 = p_ref[4]
    be0 = p_ref[5]
    be1 = p_ref[6]

    cf = cf_ref[0, :, :]
    qm = q_ref[0]

    xy2 = jnp.dot(cf, qm, preferred_element_type=jnp.float32)
    xy = xy2[0:1, :] + xy2[1:2, :]
    x_row = xy[:, :N]
    y_row = xy[:, N:]
    xcol = jnp.transpose(x_row)
    ycol = jnp.transpose(y_row)

    dx = xcol - x_row
    dy = ycol - y_row
    d = jnp.sqrt(dx * dx + dy * dy)
    xev_ref[0, :, :] = d

    s_row = x_row * wc0 + y_row * wc1 + emb1
    st_col = xcol * wc0 + ycol * wc1 + emb1
    base = st_col + s_row
    a0 = d * we0 + be0
    a1 = d * we1 + be1
    p0 = a0 + base
    p1 = a1 + base
    for t in range(NT):
        yp_ref[0, :, 2 * t, :] = p0[:, 128 * t:128 * (t + 1)]
        yp_ref[0, :, 2 * t + 1, :] = p1[:, 128 * t:128 * (t + 1)]

    tour = tour_ref[0, 0, :]
    tnext = tnext_ref[0, 0, :]
    col = jax.lax.broadcasted_iota(jnp.int32, (N, N), 1)
    a_oh = (tour[:, None] == col).astype(jnp.bfloat16)
    bn_oh = (tnext[:, None] == col).astype(jnp.bfloat16)
    dn = (((0,), (0,)), ((), ()))
    m_fwd = jax.lax.dot_general(a_oh, bn_oh, dn,
                                preferred_element_type=jnp.float32)
    m_bwd = jax.lax.dot_general(bn_oh, a_oh, dn,
                                preferred_element_type=jnp.float32)
    dd = a1 - a0
    sel = jnp.where((m_fwd + m_bwd) > 0.0, dd, 0.0) - jnp.log1p(jnp.exp(dd))
    lsum_ref[0, 0, 0] = jnp.sum(sel)


@functools.partial(jax.jit, static_argnames=("interpret",))
def kernel(x_nodes_coord, y_tour, w_coord, emb, w_e, b_e, interpret=False):
    cf = x_nodes_coord.reshape(B, 1, 2 * N)
    cf_bits = jax.lax.bitcast_convert_type(cf, jnp.uint32)
    cf_hi = jax.lax.bitcast_convert_type(
        cf_bits & jnp.uint32(0xFFFF0000), jnp.float32)
    cf2 = jnp.concatenate([cf_hi, cf - cf_hi], axis=1)
    tour = y_tour.reshape(B, 1, N)
    tnext = jnp.roll(y_tour, -1, axis=-1).reshape(B, 1, N)
    c2 = jnp.arange(2 * N, dtype=jnp.int32)
    jn = jnp.arange(N, dtype=jnp.int32)
    q = jnp.concatenate(
        [(c2[:, None] == 2 * jn[None, :]),
         (c2[:, None] == 2 * jn[None, :] + 1)], axis=1)
    q = q.astype(jnp.float32).reshape(1, 2 * N, 2 * N)
    params = jnp.stack([w_coord[0], w_coord[1], emb[1],
                        w_e[0], w_e[1], b_e[0], b_e[1]])

    yp, xev, lsum = pl.pallas_call(
        _fused_kernel,
        grid=(B,),
        in_specs=[pl.BlockSpec((1, 2, 2 * N), lambda b: (b, 0, 0)),
                  pl.BlockSpec((1, 1, N), lambda b: (b, 0, 0)),
                  pl.BlockSpec((1, 1, N), lambda b: (b, 0, 0)),
                  pl.BlockSpec((1, 2 * N, 2 * N), lambda b: (0, 0, 0)),
                  pl.BlockSpec(memory_space=pltpu.SMEM)],
        out_specs=[
            pl.BlockSpec((1, N, 2 * NT, 128), lambda b: (b, 0, 0, 0)),
            pl.BlockSpec((1, N, N), lambda b: (b, 0, 0)),
            pl.BlockSpec((1, 1, 1), lambda b: (b, 0, 0),
                         memory_space=pltpu.SMEM),
        ],
        out_shape=[
            jax.ShapeDtypeStruct((B, N, 2 * NT, 128), jnp.float32),
            jax.ShapeDtypeStruct((B, N, N), jnp.float32),
            jax.ShapeDtypeStruct((B, 1, 1), jnp.float32),
        ],
        interpret=interpret,
    )(cf2, tour, tnext, q, params)

    y_preds = (yp.reshape(B, N, NT, 2, 128)
               .transpose(0, 1, 2, 4, 3)
               .reshape(B, N, N, 2))
    loss = -jnp.sum(lsum) / jnp.float32(B * N * N)
    return (y_preds, loss, xev)


# speedup vs baseline: 1.0360x; 1.0064x over previous
import functools

import jax
import jax.numpy as jnp
from jax.experimental import pallas as pl
from jax.experimental.pallas import tpu as pltpu

B, N = 32, 512
NT = N // 128


def _fused_kernel(cf_ref, tour_ref, tnext_ref, q_ref, p_ref,
                  yp_ref, xev_ref, lsum_ref):
    wc0 = p_ref[0]
    wc1 = p_ref[1]
    emb1 = p_ref[2]
    we0 = p_ref[3]
    we1 = p_ref[4]
    be0 = p_ref[5]
    be1 = p_ref[6]

    cf = cf_ref[0, :, :]
    qm = q_ref[0]

    xy2 = jnp.dot(cf, qm, preferred_element_type=jnp.float32)
    xy = xy2[0:1, :] + xy2[1:2, :]
    x_row = xy[:, :N]
    y_row = xy[:, N:]
    xcol = jnp.transpose(x_row)
    ycol = jnp.transpose(y_row)

    dx = xcol - x_row
    dy = ycol - y_row
    d = jnp.sqrt(dx * dx + dy * dy)
    xev_ref[0, :, :] = d

    s_row = x_row * wc0 + y_row * wc1 + emb1
    st_col = xcol * wc0 + ycol * wc1 + emb1
    base = st_col + s_row
    a0 = d * we0 + be0
    a1 = d * we1 + be1
    p0 = a0 + base
    p1 = a1 + base
    for t in range(NT):
        yp_ref[0, :, 2 * t, :] = p0[:, 128 * t:128 * (t + 1)]
        yp_ref[0, :, 2 * t + 1, :] = p1[:, 128 * t:128 * (t + 1)]

    tour = tour_ref[0, 0, :]
    tnext = tnext_ref[0, 0, :]
    col = jax.lax.broadcasted_iota(jnp.int32, (N, N), 1)
    a_oh = (tour[:, None] == col).astype(jnp.bfloat16)
    bn_oh = (tnext[:, None] == col).astype(jnp.bfloat16)
    dn = (((0,), (0,)), ((), ()))
    m_fwd = jax.lax.dot_general(a_oh, bn_oh, dn,
                                preferred_element_type=jnp.float32)
    m_bwd = jax.lax.dot_general(bn_oh, a_oh, dn,
                                preferred_element_type=jnp.float32)
    dd = a1 - a0
    sel = jnp.where((m_fwd + m_bwd) > 0.0, dd, 0.0) - jnp.log1p(jnp.exp(dd))
    lsum_ref[0, 0, 0] = jnp.sum(sel)


@functools.partial(jax.jit, static_argnames=("interpret",))
def kernel(x_nodes_coord, y_tour, w_coord, emb, w_e, b_e, interpret=False):
    cf = x_nodes_coord.reshape(B, 1, 2 * N)
    cf_bits = jax.lax.bitcast_convert_type(cf, jnp.uint32)
    cf_hi = jax.lax.bitcast_convert_type(
        cf_bits & jnp.uint32(0xFFFF0000), jnp.float32)
    cf2 = jnp.concatenate([cf_hi, cf - cf_hi], axis=1)
    cf2 = cf2.astype(jnp.bfloat16)
    tour = y_tour.reshape(B, 1, N)
    tnext = jnp.roll(y_tour, -1, axis=-1).reshape(B, 1, N)
    c2 = jnp.arange(2 * N, dtype=jnp.int32)
    jn = jnp.arange(N, dtype=jnp.int32)
    q = jnp.concatenate(
        [(c2[:, None] == 2 * jn[None, :]),
         (c2[:, None] == 2 * jn[None, :] + 1)], axis=1)
    q = q.astype(jnp.bfloat16).reshape(1, 2 * N, 2 * N)
    params = jnp.stack([w_coord[0], w_coord[1], emb[1],
                        w_e[0], w_e[1], b_e[0], b_e[1]])

    yp, xev, lsum = pl.pallas_call(
        _fused_kernel,
        grid=(B,),
        in_specs=[pl.BlockSpec((1, 2, 2 * N), lambda b: (b, 0, 0)),
                  pl.BlockSpec((1, 1, N), lambda b: (b, 0, 0)),
                  pl.BlockSpec((1, 1, N), lambda b: (b, 0, 0)),
                  pl.BlockSpec((1, 2 * N, 2 * N), lambda b: (0, 0, 0)),
                  pl.BlockSpec(memory_space=pltpu.SMEM)],
        out_specs=[
            pl.BlockSpec((1, N, 2 * NT, 128), lambda b: (b, 0, 0, 0)),
            pl.BlockSpec((1, N, N), lambda b: (b, 0, 0)),
            pl.BlockSpec((1, 1, 1), lambda b: (b, 0, 0),
                         memory_space=pltpu.SMEM),
        ],
        out_shape=[
            jax.ShapeDtypeStruct((B, N, 2 * NT, 128), jnp.float32),
            jax.ShapeDtypeStruct((B, N, N), jnp.float32),
            jax.ShapeDtypeStruct((B, 1, 1), jnp.float32),
        ],
        interpret=interpret,
    )(cf2, tour, tnext, q, params)

    y_preds = (yp.reshape(B, N, NT, 2, 128)
               .transpose(0, 1, 2, 4, 3)
               .reshape(B, N, N, 2))
    loss = -jnp.sum(lsum) / jnp.float32(B * N * N)
    return (y_preds, loss, xev)
